# baseline (device time: 73411 ns/iter reference)
import jax
import jax.numpy as jnp
from jax import lax
from jax.experimental import pallas as pl
from jax.experimental.pallas import tpu as pltpu

M = 8192
N = 2048
N_GLOBAL = 4096
BM = 1024
K = M // BM
SUB = BM // 128
PACK = M // 128
EPS = 1e-5


def kernel(x, gamma, beta):
    def body(x_ref, g_ref, b_ref, o_ref, xstash, s_send, s_recv, s_fin,
             send_sem, recv_sem):
        i = pl.program_id(0)
        my_x = lax.axis_index("x")
        my_y = lax.axis_index("y")
        partner = (my_x, 1 - my_y)

        f32 = jnp.float32
        bf16 = jnp.bfloat16
        row_div = lax.broadcasted_iota(jnp.int32, (BM, SUB), 0) // 128
        grp = lax.broadcasted_iota(jnp.int32, (BM, SUB), 1)
        sel_grp = (row_div == grp).astype(bf16)
        row_mod = lax.broadcasted_iota(jnp.int32, (BM, 128), 0) % 128
        lane = lax.broadcasted_iota(jnp.int32, (BM, 128), 1)
        sel_lane = (row_mod == lane).astype(bf16)

        def pack(col):
            masked = col.astype(bf16) * sel_lane
            return jnp.dot(sel_grp.T, masked, preferred_element_type=f32)

        def unpack(p):
            exp = jnp.dot(sel_grp, p.astype(bf16), preferred_element_type=f32)
            return jnp.sum(exp * sel_lane.astype(f32), axis=1, keepdims=True)

        @pl.when(i == 0)
        def _barrier():
            bar = pltpu.get_barrier_semaphore()
            pl.semaphore_signal(bar, inc=1, device_id=partner,
                                device_id_type=pl.DeviceIdType.MESH)
            pl.semaphore_wait(bar, 1)

        @pl.when(i < K)
        def _stats():
            xb = x_ref[...]
            xh = xb.astype(bf16)
            ones = jnp.ones((N, 1), dtype=bf16)
            s1 = jnp.dot(xh, ones, preferred_element_type=f32)
            s2 = jnp.dot(xh * xh, ones, preferred_element_type=f32)
            s_send[pl.ds(i * SUB, SUB), :] = pack(s1)
            s_send[pl.ds(PACK + i * SUB, SUB), :] = pack(s2)
            xstash[pl.ds(i * BM, BM), :] = xh

        @pl.when(i == K - 1)
        def _exchange():
            rdma = pltpu.make_async_remote_copy(
                src_ref=s_send, dst_ref=s_recv,
                send_sem=send_sem, recv_sem=recv_sem,
                device_id=partner, device_id_type=pl.DeviceIdType.MESH)
            rdma.start()
            rdma.wait()
            tot = s_send[...] + s_recv[...]
            mean = tot[:PACK, :] * (1.0 / N_GLOBAL)
            ex2 = tot[PACK:, :] * (1.0 / N_GLOBAL)
            var = jnp.maximum(ex2 - mean * mean, 0.0)
            s_fin[:PACK, :] = mean
            s_fin[PACK:, :] = lax.rsqrt(var + EPS)

        @pl.when(i >= K)
        def _normalize():
            j = i - K
            xb = xstash[pl.ds(j * BM, BM), :].astype(jnp.float32)
            mean = unpack(s_fin[pl.ds(j * SUB, SUB), :])
            rstd = unpack(s_fin[pl.ds(PACK + j * SUB, SUB), :])
            g = g_ref[...].reshape(1, N)
            b = b_ref[...].reshape(1, N)
            nmr = -mean * rstd
            t = xb * rstd + nmr
            o_ref[...] = (t * g + b).astype(bf16)

    return pl.pallas_call(
        body,
        grid=(2 * K,),
        in_specs=[
            pl.BlockSpec((BM, N), lambda i: (jnp.minimum(i, K - 1), 0)),
            pl.BlockSpec((N,), lambda i: (0,)),
            pl.BlockSpec((N,), lambda i: (0,)),
        ],
        out_specs=pl.BlockSpec(
            (BM, N), lambda i: (jnp.where(i < K, 0, i - K), 0)
        ),
        out_shape=jax.ShapeDtypeStruct((M, N), jnp.bfloat16),
        scratch_shapes=[
            pltpu.VMEM((M, N), jnp.bfloat16),
            pltpu.VMEM((2 * PACK, 128), jnp.float32),
            pltpu.VMEM((2 * PACK, 128), jnp.float32),
            pltpu.VMEM((2 * PACK, 128), jnp.float32),
            pltpu.SemaphoreType.DMA,
            pltpu.SemaphoreType.DMA,
        ],
        compiler_params=pltpu.CompilerParams(
            collective_id=0,
            dimension_semantics=("arbitrary",),
            vmem_limit_bytes=96 * 1024 * 1024,
        ),
    )(x, gamma, beta)


# device time: 61011 ns/iter; 1.2032x vs baseline; 1.2032x over previous
import jax
import jax.numpy as jnp
from jax import lax
from jax.experimental import pallas as pl
from jax.experimental.pallas import tpu as pltpu

M = 8192
N = 2048
N_GLOBAL = 4096
BM = 1024
K = M // BM
SUB = BM // 128
PACK = M // 128
EPS = 1e-5


def kernel(x, gamma, beta):
    def body(x_ref, g_ref, b_ref, o_ref, xstash, s_send, s_recv, s_fin,
             selg_ref, sell_ref, send_sem, recv_sem):
        i = pl.program_id(0)
        my_x = lax.axis_index("x")
        my_y = lax.axis_index("y")
        partner = (my_x, 1 - my_y)

        f32 = jnp.float32
        bf16 = jnp.bfloat16

        @pl.when(i == 0)
        def _init_onehots():
            row_div = lax.broadcasted_iota(jnp.int32, (BM, SUB), 0) // 128
            grp = lax.broadcasted_iota(jnp.int32, (BM, SUB), 1)
            selg_ref[...] = (row_div == grp).astype(f32)
            row_mod = lax.broadcasted_iota(jnp.int32, (BM, 128), 0) % 128
            lane = lax.broadcasted_iota(jnp.int32, (BM, 128), 1)
            sell_ref[...] = (row_mod == lane).astype(f32)

        def pack(col):
            return jnp.dot(selg_ref[...].T, col * sell_ref[...],
                           preferred_element_type=f32)

        def unpack(p):
            exp = jnp.dot(selg_ref[...].astype(bf16), p.astype(bf16),
                          preferred_element_type=f32)
            return jnp.sum(exp * sell_ref[...], axis=1, keepdims=True)

        @pl.when(i == 0)
        def _barrier():
            bar = pltpu.get_barrier_semaphore()
            pl.semaphore_signal(bar, inc=1, device_id=partner,
                                device_id_type=pl.DeviceIdType.MESH)
            pl.semaphore_wait(bar, 1)

        @pl.when(i < K)
        def _stats():
            xb = x_ref[...]
            s1 = jnp.sum(xb, axis=1, keepdims=True)
            s2 = jnp.sum(xb * xb, axis=1, keepdims=True)
            s_send[pl.ds(i * SUB, SUB), :] = pack(s1)
            s_send[pl.ds(PACK + i * SUB, SUB), :] = pack(s2)
            xstash[pl.ds(i * BM, BM), :] = xb.astype(bf16)

        @pl.when(i == K - 1)
        def _exchange():
            rdma = pltpu.make_async_remote_copy(
                src_ref=s_send, dst_ref=s_recv,
                send_sem=send_sem, recv_sem=recv_sem,
                device_id=partner, device_id_type=pl.DeviceIdType.MESH)
            rdma.start()
            rdma.wait()
            tot = s_send[...] + s_recv[...]
            mean = tot[:PACK, :] * (1.0 / N_GLOBAL)
            ex2 = tot[PACK:, :] * (1.0 / N_GLOBAL)
            var = jnp.maximum(ex2 - mean * mean, 0.0)
            s_fin[:PACK, :] = mean
            s_fin[PACK:, :] = lax.rsqrt(var + EPS)

        @pl.when(i >= K)
        def _normalize():
            j = i - K
            xh = xstash[pl.ds(j * BM, BM), :]
            mean = unpack(s_fin[pl.ds(j * SUB, SUB), :])
            rstd = unpack(s_fin[pl.ds(PACK + j * SUB, SUB), :])
            nmr_h = (-mean * rstd).astype(bf16)
            rstd_h = rstd.astype(bf16)
            g = g_ref[...].reshape(1, N).astype(bf16)
            b = b_ref[...].reshape(1, N).astype(bf16)
            t = xh * rstd_h + nmr_h
            o_ref[...] = t * g + b

    return pl.pallas_call(
        body,
        grid=(2 * K,),
        in_specs=[
            pl.BlockSpec((BM, N), lambda i: (jnp.minimum(i, K - 1), 0)),
            pl.BlockSpec((N,), lambda i: (0,)),
            pl.BlockSpec((N,), lambda i: (0,)),
        ],
        out_specs=pl.BlockSpec(
            (BM, N), lambda i: (jnp.where(i < K, 0, i - K), 0)
        ),
        out_shape=jax.ShapeDtypeStruct((M, N), jnp.bfloat16),
        scratch_shapes=[
            pltpu.VMEM((M, N), jnp.bfloat16),
            pltpu.VMEM((2 * PACK, 128), jnp.float32),
            pltpu.VMEM((2 * PACK, 128), jnp.float32),
            pltpu.VMEM((2 * PACK, 128), jnp.float32),
            pltpu.VMEM((BM, SUB), jnp.float32),
            pltpu.VMEM((BM, 128), jnp.float32),
            pltpu.SemaphoreType.DMA,
            pltpu.SemaphoreType.DMA,
        ],
        compiler_params=pltpu.CompilerParams(
            collective_id=0,
            dimension_semantics=("arbitrary",),
            vmem_limit_bytes=96 * 1024 * 1024,
        ),
    )(x, gamma, beta)
